# baseline (speedup 1.0000x reference)
import numpy as np
import jax
import jax.numpy as jnp
from jax.experimental import pallas as pl
from jax.experimental.pallas import tpu as pltpu

_MAX_DIST = 18
_BOARD = 19
_N = _BOARD * _BOARD
_BLOCKS = 4


def _build_code_map():
    dirs = [(-1, 0), (1, 0), (0, -1), (0, 1),
            (-1, -1), (-1, 1), (1, -1), (1, 1)]
    tm = np.zeros((_N, _N), np.int8)
    rr, cc = np.meshgrid(np.arange(_BOARD), np.arange(_BOARD), indexing="ij")
    p_flat = rr * _BOARD + cc
    for d, (dr, dc) in enumerate(dirs):
        off = 0 if d < 4 else _MAX_DIST
        for t in range(1, _MAX_DIST + 1):
            tr = rr + dr * t
            tc = cc + dc * t
            valid = (tr >= 0) & (tr < _BOARD) & (tc >= 0) & (tc < _BOARD)
            p = p_flat[valid]
            q = (tr * _BOARD + tc)[valid]
            tm[q, p] = off + t
    return tm


_TM_NP = _build_code_map()


def _body(w_ref, tm_ref, x_ref, out_ref, mt_ref):
    @pl.when(pl.program_id(0) == 0)
    def _build():
        tm = tm_ref[...].astype(jnp.int32)
        mt = jnp.zeros((_N, _N), jnp.float32)
        for t in range(1, _MAX_DIST + 1):
            mt = mt + jnp.where(tm == t, w_ref[0, t - 1], 0.0)
            mt = mt + jnp.where(tm == _MAX_DIST + t, w_ref[1, t - 1], 0.0)
        mt_ref[...] = mt

    out_ref[...] = jnp.dot(x_ref[...], mt_ref[...],
                           preferred_element_type=jnp.float32)


def kernel(x, weight):
    B, C, H, W = x.shape
    n = B * C
    blk = n // _BLOCKS
    xf = x.reshape(n, H * W)
    out = pl.pallas_call(
        _body,
        grid=(_BLOCKS,),
        out_shape=jax.ShapeDtypeStruct((n, H * W), jnp.float32),
        in_specs=[
            pl.BlockSpec(memory_space=pltpu.SMEM),
            pl.BlockSpec((_N, _N), lambda i: (0, 0)),
            pl.BlockSpec((blk, H * W), lambda i: (i, 0)),
        ],
        out_specs=pl.BlockSpec((blk, H * W), lambda i: (i, 0)),
        scratch_shapes=[pltpu.VMEM((_N, _N), jnp.float32)],
    )(weight, jnp.asarray(_TM_NP), xf)
    return out.reshape(B, C, H, W)


# speedup vs baseline: 1.1023x; 1.1023x over previous
import numpy as np
import jax
import jax.numpy as jnp
from jax.experimental import pallas as pl
from jax.experimental.pallas import tpu as pltpu

_MAX_DIST = 18
_BOARD = 19
_N = _BOARD * _BOARD


def _build_code_map():
    dirs = [(-1, 0), (1, 0), (0, -1), (0, 1),
            (-1, -1), (-1, 1), (1, -1), (1, 1)]
    tm = np.zeros((_N, _N), np.int8)
    rr, cc = np.meshgrid(np.arange(_BOARD), np.arange(_BOARD), indexing="ij")
    p_flat = rr * _BOARD + cc
    for d, (dr, dc) in enumerate(dirs):
        off = 0 if d < 4 else _MAX_DIST
        for t in range(1, _MAX_DIST + 1):
            tr = rr + dr * t
            tc = cc + dc * t
            valid = (tr >= 0) & (tr < _BOARD) & (tc >= 0) & (tc < _BOARD)
            p = p_flat[valid]
            q = (tr * _BOARD + tc)[valid]
            tm[q, p] = off + t
    return tm


_TM_NP = _build_code_map()


def _body(w_ref, tm_ref, x_ref, out_ref):
    tm = tm_ref[...].astype(jnp.int32)
    mt = jnp.zeros((_N, _N), jnp.float32)
    for t in range(1, _MAX_DIST + 1):
        mt = mt + jnp.where(tm == t, w_ref[0, t - 1], 0.0)
        mt = mt + jnp.where(tm == _MAX_DIST + t, w_ref[1, t - 1], 0.0)
    out_ref[...] = jnp.dot(x_ref[...], mt, preferred_element_type=jnp.float32)


def kernel(x, weight):
    B, C, H, W = x.shape
    xf = x.reshape(B * C, H * W)
    out = pl.pallas_call(
        _body,
        out_shape=jax.ShapeDtypeStruct((B * C, H * W), jnp.float32),
        in_specs=[
            pl.BlockSpec(memory_space=pltpu.SMEM),
            pl.BlockSpec(memory_space=pltpu.VMEM),
            pl.BlockSpec(memory_space=pltpu.VMEM),
        ],
        out_specs=pl.BlockSpec(memory_space=pltpu.VMEM),
    )(weight, jnp.asarray(_TM_NP), xf)
    return out.reshape(B, C, H, W)
